# Initial kernel scaffold; baseline (speedup 1.0000x reference)
import jax
import jax.numpy as jnp
from jax.experimental import pallas as pl


def kernel(input_D):
    raise NotImplementedError("write your pallas kernel here")


# speedup vs baseline: 31.8528x; 31.8528x over previous
import functools

import jax
import jax.numpy as jnp
from jax import lax
from jax.experimental import pallas as pl
from jax.experimental.pallas import tpu as pltpu

B = 512


def _cumsum_kernel(d_ref, sz_ref, sc2_ref, dd_ref, ddm1_ref, e_ref,
                   rowcarry, colcarry, colprev, prevdiag, ecorner):
    i = pl.program_id(0)
    j = pl.program_id(1)
    X = d_ref[...]

    iota_r = lax.broadcasted_iota(jnp.int32, (B, B), 0)
    iota_c = lax.broadcasted_iota(jnp.int32, (B, B), 1)
    U = (iota_r <= iota_c).astype(jnp.float32)
    L = (iota_r >= iota_c).astype(jnp.float32)

    rc = lax.dot_general(X, U, (((1,), (0,)), ((), ())),
                         preferred_element_type=jnp.float32)
    rc = rc + jnp.where(j > 0, rowcarry[...], 0.0)
    rowcarry[...] = rc[:, B - 1:B]

    cc = lax.dot_general(L, rc, (((1,), (0,)), ((), ())),
                         preferred_element_type=jnp.float32)
    cc_top = jnp.where(i > 0, colcarry[0:1, pl.ds(j * B, B)], 0.0)
    S_blk = cc + cc_top

    sz_ref[...] = jnp.concatenate([cc_top, S_blk[:B - 1, :]], axis=0)
    cp = jnp.where(j > 0, colprev[...], 0.0)
    sc2_ref[...] = jnp.concatenate([cp, S_blk[:, :B - 1]], axis=1)

    colprev[...] = S_blk[:, B - 1:B]
    colcarry[0:1, pl.ds(j * B, B)] = S_blk[B - 1:B, :]

    @pl.when(i == j)
    def _diag():
        eye = (iota_r == iota_c).astype(jnp.float32)
        ddrow = jnp.sum(S_blk * eye, axis=0, keepdims=True)
        dd_ref[0:1, pl.ds(i * B, B)] = ddrow
        pd = jnp.where(i > 0, prevdiag[...], 0.0)
        ddcol = jnp.sum(S_blk * eye, axis=1, keepdims=True)
        ddm1_ref[pl.ds(i * B, B), 0:1] = jnp.concatenate(
            [pd, ddcol[:B - 1, :]], axis=0)
        prevdiag[...] = ddrow[:, B - 1:B]
        shift = (iota_c == iota_r + 1).astype(jnp.float32)
        tmpc = jnp.sum(X * shift, axis=1, keepdims=True)
        ec = jnp.where(i > 0, ecorner[...], 0.0)
        e_ref[pl.ds(i * B, B), 0:1] = jnp.concatenate(
            [ec, tmpc[:B - 1, :]], axis=0)

    @pl.when(j == i + 1)
    def _corner():
        mask = jnp.logical_and(iota_r == B - 1, iota_c == 0)
        ecorner[...] = jnp.sum(jnp.where(mask, X, 0.0), axis=(0, 1),
                               keepdims=True)


def _out_kernel(sz_ref, sc2_ref, dd_ref, ddm1_ref, e_ref, o_ref):
    i = pl.program_id(0)
    j = pl.program_id(1)
    a = jnp.minimum(i, j)
    b = jnp.maximum(i, j)

    Z = sz_ref[...]
    M = sc2_ref[...]
    W = M.T
    ddc = dd_ref[0:1, pl.ds(b * B, B)]
    ddr = ddm1_ref[pl.ds(a * B, B), 0:1]

    Ublk = ddc + ddr - Z - W
    UblkT = Ublk.T

    iota_r = lax.broadcasted_iota(jnp.int32, (B, B), 0)
    iota_c = lax.broadcasted_iota(jnp.int32, (B, B), 1)
    rg = i * B + iota_r
    cg = j * B + iota_c
    out = jnp.where(cg >= rg, Ublk, UblkT)
    esel = e_ref[pl.ds(i * B, B), 0:1]
    out = jnp.where(rg == cg + 1, esel, out)
    o_ref[...] = out


@functools.partial(jax.jit, static_argnames=("interpret",))
def kernel(input_D, interpret=False):
    D = input_D[0]
    n = D.shape[0]
    g = n // B

    sz, sc2, dd, ddm1, e = pl.pallas_call(
        _cumsum_kernel,
        grid=(g, g),
        in_specs=[pl.BlockSpec((B, B), lambda i, j: (i, j))],
        out_specs=[
            pl.BlockSpec((B, B), lambda i, j: (i, j)),
            pl.BlockSpec((B, B), lambda i, j: (i, j)),
            pl.BlockSpec((1, n), lambda i, j: (0, 0)),
            pl.BlockSpec((n, 1), lambda i, j: (0, 0)),
            pl.BlockSpec((n, 1), lambda i, j: (0, 0)),
        ],
        out_shape=[
            jax.ShapeDtypeStruct((n, n), jnp.float32),
            jax.ShapeDtypeStruct((n, n), jnp.float32),
            jax.ShapeDtypeStruct((1, n), jnp.float32),
            jax.ShapeDtypeStruct((n, 1), jnp.float32),
            jax.ShapeDtypeStruct((n, 1), jnp.float32),
        ],
        scratch_shapes=[
            pltpu.VMEM((B, 1), jnp.float32),
            pltpu.VMEM((1, n), jnp.float32),
            pltpu.VMEM((B, 1), jnp.float32),
            pltpu.VMEM((1, 1), jnp.float32),
            pltpu.VMEM((1, 1), jnp.float32),
        ],
        compiler_params=pltpu.CompilerParams(
            dimension_semantics=("arbitrary", "arbitrary")),
        interpret=interpret,
    )(D)

    out = pl.pallas_call(
        _out_kernel,
        grid=(g, g),
        in_specs=[
            pl.BlockSpec((B, B), lambda i, j: (jnp.minimum(i, j),
                                               jnp.maximum(i, j))),
            pl.BlockSpec((B, B), lambda i, j: (jnp.maximum(i, j),
                                               jnp.minimum(i, j))),
            pl.BlockSpec((1, n), lambda i, j: (0, 0)),
            pl.BlockSpec((n, 1), lambda i, j: (0, 0)),
            pl.BlockSpec((n, 1), lambda i, j: (0, 0)),
        ],
        out_specs=pl.BlockSpec((B, B), lambda i, j: (i, j)),
        out_shape=jax.ShapeDtypeStruct((n, n), jnp.float32),
        compiler_params=pltpu.CompilerParams(
            dimension_semantics=("arbitrary", "arbitrary")),
        interpret=interpret,
    )(sz, sc2, dd, ddm1, e)

    return out[None, :, :]


# speedup vs baseline: 32.2054x; 1.0111x over previous
import functools

import jax
import jax.numpy as jnp
from jax import lax
from jax.experimental import pallas as pl
from jax.experimental.pallas import tpu as pltpu

B = 512
NLANE = 16


def _cumsum_kernel(d_ref, s_ref, brow_ref, bcol_ref, dd_ref, ddm1_ref, e_ref,
                   rowcarry, colcarry, prevdiag, ecorner):
    i = pl.program_id(0)
    j = pl.program_id(1)
    X = d_ref[...]

    iota_r = lax.broadcasted_iota(jnp.int32, (B, B), 0)
    iota_c = lax.broadcasted_iota(jnp.int32, (B, B), 1)
    U = (iota_r <= iota_c).astype(jnp.float32)
    L = (iota_r >= iota_c).astype(jnp.float32)

    rc = lax.dot_general(X, U, (((1,), (0,)), ((), ())),
                         preferred_element_type=jnp.float32)
    rc = rc + jnp.where(j > 0, rowcarry[...], 0.0)
    rowcarry[...] = rc[:, B - 1:B]

    cc = lax.dot_general(L, rc, (((1,), (0,)), ((), ())),
                         preferred_element_type=jnp.float32)
    cc_top = jnp.where(i > 0, colcarry[0:1, pl.ds(j * B, B)], 0.0)
    S_blk = cc + cc_top
    s_ref[...] = S_blk

    colcarry[0:1, pl.ds(j * B, B)] = S_blk[B - 1:B, :]
    brow_ref[pl.ds((i + 1) * 8, 1), pl.ds(j * B, B)] = S_blk[B - 1:B, :]
    lane16 = lax.broadcasted_iota(jnp.int32, (B, NLANE), 1)
    cur = bcol_ref[pl.ds(i * B, B), 0:NLANE]
    bcol_ref[pl.ds(i * B, B), 0:NLANE] = jnp.where(
        lane16 == j + 1, S_blk[:, B - 1:B], cur)

    @pl.when(i == j)
    def _diag():
        eye = (iota_r == iota_c).astype(jnp.float32)
        ddrow = jnp.sum(S_blk * eye, axis=0, keepdims=True)
        dd_ref[0:1, pl.ds(i * B, B)] = ddrow
        pd = jnp.where(i > 0, prevdiag[...], 0.0)
        ddcol = jnp.sum(S_blk * eye, axis=1, keepdims=True)
        ddm1_ref[pl.ds(i * B, B), 0:1] = jnp.concatenate(
            [pd, ddcol[:B - 1, :]], axis=0)
        prevdiag[...] = ddrow[:, B - 1:B]
        shift = (iota_c == iota_r + 1).astype(jnp.float32)
        tmpc = jnp.sum(X * shift, axis=1, keepdims=True)
        ec = jnp.where(i > 0, ecorner[...], 0.0)
        e_ref[pl.ds(i * B, B), 0:1] = jnp.concatenate(
            [ec, tmpc[:B - 1, :]], axis=0)

    @pl.when(j == i + 1)
    def _corner():
        mask = jnp.logical_and(iota_r == B - 1, iota_c == 0)
        ecorner[...] = jnp.sum(jnp.where(mask, X, 0.0), axis=(0, 1),
                               keepdims=True)


def _out_kernel(sab_ref, sba_ref, brow_ref, bcol_ref, dd_ref, ddm1_ref, e_ref,
                o_ref):
    i = pl.program_id(0)
    j = pl.program_id(1)
    a = jnp.minimum(i, j)
    b = jnp.maximum(i, j)

    Sab = sab_ref[...]
    Sba = sba_ref[...]

    brow = brow_ref[pl.ds(a * 8, 8), pl.ds(b * B, B)][0:1, :]
    brow = jnp.where(a > 0, brow, 0.0)
    Z = jnp.concatenate([brow, Sab[:B - 1, :]], axis=0)

    lane16 = lax.broadcasted_iota(jnp.int32, (B, NLANE), 1)
    bc_blk = bcol_ref[pl.ds(b * B, B), 0:NLANE]
    bcol = jnp.sum(jnp.where(lane16 == a, bc_blk, 0.0), axis=1, keepdims=True)
    bcol = jnp.where(a > 0, bcol, 0.0)
    M_sh = jnp.concatenate([bcol, Sba[:, :B - 1]], axis=1)
    W = M_sh.T

    ddc = dd_ref[0:1, pl.ds(b * B, B)]
    ddr = ddm1_ref[pl.ds(a * B, B), 0:1]

    Ublk = ddc + ddr - Z - W
    UblkT = Ublk.T

    iota_r = lax.broadcasted_iota(jnp.int32, (B, B), 0)
    iota_c = lax.broadcasted_iota(jnp.int32, (B, B), 1)
    rg = i * B + iota_r
    cg = j * B + iota_c
    out = jnp.where(cg >= rg, Ublk, UblkT)
    esel = e_ref[pl.ds(i * B, B), 0:1]
    out = jnp.where(rg == cg + 1, esel, out)
    o_ref[...] = out


@functools.partial(jax.jit, static_argnames=("interpret",))
def kernel(input_D, interpret=False):
    D = input_D[0]
    n = D.shape[0]
    g = n // B

    s, brow, bcol, dd, ddm1, e = pl.pallas_call(
        _cumsum_kernel,
        grid=(g, g),
        in_specs=[pl.BlockSpec((B, B), lambda i, j: (i, j))],
        out_specs=[
            pl.BlockSpec((B, B), lambda i, j: (i, j)),
            pl.BlockSpec((8 * (g + 1), n), lambda i, j: (0, 0)),
            pl.BlockSpec((n, NLANE), lambda i, j: (0, 0)),
            pl.BlockSpec((1, n), lambda i, j: (0, 0)),
            pl.BlockSpec((n, 1), lambda i, j: (0, 0)),
            pl.BlockSpec((n, 1), lambda i, j: (0, 0)),
        ],
        out_shape=[
            jax.ShapeDtypeStruct((n, n), jnp.float32),
            jax.ShapeDtypeStruct((8 * (g + 1), n), jnp.float32),
            jax.ShapeDtypeStruct((n, NLANE), jnp.float32),
            jax.ShapeDtypeStruct((1, n), jnp.float32),
            jax.ShapeDtypeStruct((n, 1), jnp.float32),
            jax.ShapeDtypeStruct((n, 1), jnp.float32),
        ],
        scratch_shapes=[
            pltpu.VMEM((B, 1), jnp.float32),
            pltpu.VMEM((1, n), jnp.float32),
            pltpu.VMEM((1, 1), jnp.float32),
            pltpu.VMEM((1, 1), jnp.float32),
        ],
        compiler_params=pltpu.CompilerParams(
            dimension_semantics=("arbitrary", "arbitrary")),
        interpret=interpret,
    )(D)

    out = pl.pallas_call(
        _out_kernel,
        grid=(g, g),
        in_specs=[
            pl.BlockSpec((B, B), lambda i, j: (jnp.minimum(i, j),
                                               jnp.maximum(i, j))),
            pl.BlockSpec((B, B), lambda i, j: (jnp.maximum(i, j),
                                               jnp.minimum(i, j))),
            pl.BlockSpec((8 * (g + 1), n), lambda i, j: (0, 0)),
            pl.BlockSpec((n, NLANE), lambda i, j: (0, 0)),
            pl.BlockSpec((1, n), lambda i, j: (0, 0)),
            pl.BlockSpec((n, 1), lambda i, j: (0, 0)),
            pl.BlockSpec((n, 1), lambda i, j: (0, 0)),
        ],
        out_specs=pl.BlockSpec((B, B), lambda i, j: (i, j)),
        out_shape=jax.ShapeDtypeStruct((n, n), jnp.float32),
        compiler_params=pltpu.CompilerParams(
            dimension_semantics=("arbitrary", "arbitrary")),
        interpret=interpret,
    )(s, s, brow, bcol, dd, ddm1, e)

    return out[None, :, :]


# speedup vs baseline: 32.8731x; 1.0207x over previous
import functools

import jax
import jax.numpy as jnp
from jax import lax
from jax.experimental import pallas as pl
from jax.experimental.pallas import tpu as pltpu

B = 512
NLANE = 16


def _cumsum_kernel(d_ref, s_ref, brow_ref, bcol_ref, ddr_ref, ddc_ref,
                   dm1c_ref, dm1r_ref, e_ref,
                   rowcarry, colcarry, prevdiag, ecorner):
    i = pl.program_id(0)
    j = pl.program_id(1)
    X = d_ref[...]

    iota_r = lax.broadcasted_iota(jnp.int32, (B, B), 0)
    iota_c = lax.broadcasted_iota(jnp.int32, (B, B), 1)
    U = (iota_r <= iota_c).astype(jnp.float32)
    L = (iota_r >= iota_c).astype(jnp.float32)

    rc = lax.dot_general(X, U, (((1,), (0,)), ((), ())),
                         preferred_element_type=jnp.float32)
    rc = rc + jnp.where(j > 0, rowcarry[...], 0.0)
    rowcarry[...] = rc[:, B - 1:B]

    cc = lax.dot_general(L, rc, (((1,), (0,)), ((), ())),
                         preferred_element_type=jnp.float32)
    cc_top = jnp.where(i > 0, colcarry[0:1, pl.ds(j * B, B)], 0.0)
    S_blk = cc + cc_top
    s_ref[...] = S_blk

    colcarry[0:1, pl.ds(j * B, B)] = S_blk[B - 1:B, :]
    brow_ref[pl.ds((i + 1) * 8, 1), pl.ds(j * B, B)] = S_blk[B - 1:B, :]
    lane16 = lax.broadcasted_iota(jnp.int32, (B, NLANE), 1)
    cur = bcol_ref[pl.ds(i * B, B), 0:NLANE]
    bcol_ref[pl.ds(i * B, B), 0:NLANE] = jnp.where(
        lane16 == j + 1, S_blk[:, B - 1:B], cur)

    @pl.when(i == j)
    def _diag():
        eye = (iota_r == iota_c).astype(jnp.float32)
        ddrow = jnp.sum(S_blk * eye, axis=0, keepdims=True)
        ddcol = jnp.sum(S_blk * eye, axis=1, keepdims=True)
        ddr_ref[0:1, pl.ds(i * B, B)] = ddrow
        ddc_ref[pl.ds(i * B, B), 0:1] = ddcol
        pd = jnp.where(i > 0, prevdiag[...], 0.0)
        dm1c_ref[pl.ds(i * B, B), 0:1] = jnp.concatenate(
            [pd, ddcol[:B - 1, :]], axis=0)
        dm1r_ref[0:1, pl.ds(i * B, B)] = jnp.concatenate(
            [pd, ddrow[:, :B - 1]], axis=1)
        prevdiag[...] = ddrow[:, B - 1:B]
        shift = (iota_c == iota_r + 1).astype(jnp.float32)
        tmpc = jnp.sum(X * shift, axis=1, keepdims=True)
        ec = jnp.where(i > 0, ecorner[...], 0.0)
        e_ref[pl.ds(i * B, B), 0:1] = jnp.concatenate(
            [ec, tmpc[:B - 1, :]], axis=0)

    @pl.when(j == i + 1)
    def _corner():
        mask = jnp.logical_and(iota_r == B - 1, iota_c == 0)
        ecorner[...] = jnp.sum(jnp.where(mask, X, 0.0), axis=(0, 1),
                               keepdims=True)


def _out_kernel(sab_ref, sba_ref, brow_ref, bcol_ref, ddr_ref, ddc_ref,
                dm1c_ref, dm1r_ref, e_ref, o_ref):
    i = pl.program_id(0)
    j = pl.program_id(1)
    a = jnp.minimum(i, j)
    b = jnp.maximum(i, j)

    Sab = sab_ref[...]
    Sba = sba_ref[...]

    brow = brow_ref[pl.ds(a * 8, 8), pl.ds(b * B, B)][0:1, :]
    brow = jnp.where(a > 0, brow, 0.0)
    Z = jnp.concatenate([brow, Sab[:B - 1, :]], axis=0)

    lane16 = lax.broadcasted_iota(jnp.int32, (B, NLANE), 1)
    bc_blk = bcol_ref[pl.ds(b * B, B), 0:NLANE]
    bcol = jnp.sum(jnp.where(lane16 == a, bc_blk, 0.0), axis=1, keepdims=True)
    bcol = jnp.where(a > 0, bcol, 0.0)
    M_sh = jnp.concatenate([bcol, Sba[:, :B - 1]], axis=1)

    @pl.when(i < j)
    def _upper():
        ddc = ddr_ref[0:1, pl.ds(b * B, B)]
        ddr = dm1c_ref[pl.ds(a * B, B), 0:1]
        o_ref[...] = ddc + ddr - Z - M_sh.T

    @pl.when(i > j)
    def _lower():
        ddrv = ddc_ref[pl.ds(b * B, B), 0:1]
        ddcv = dm1r_ref[0:1, pl.ds(a * B, B)]
        out = ddrv + ddcv - Z.T - M_sh

        @pl.when(i == j + 1)
        def _corner():
            iota_r = lax.broadcasted_iota(jnp.int32, (B, B), 0)
            iota_c = lax.broadcasted_iota(jnp.int32, (B, B), 1)
            ecorn = e_ref[pl.ds(i * B, 1), 0:1]
            mask = jnp.logical_and(iota_r == 0, iota_c == B - 1)
            o_ref[...] = jnp.where(mask, ecorn, out)

        @pl.when(i != j + 1)
        def _plain():
            o_ref[...] = out

    @pl.when(i == j)
    def _diag():
        ddc = ddr_ref[0:1, pl.ds(b * B, B)]
        ddr = dm1c_ref[pl.ds(a * B, B), 0:1]
        Ublk = ddc + ddr - Z - M_sh.T
        iota_r = lax.broadcasted_iota(jnp.int32, (B, B), 0)
        iota_c = lax.broadcasted_iota(jnp.int32, (B, B), 1)
        out = jnp.where(iota_c >= iota_r, Ublk, Ublk.T)
        esel = e_ref[pl.ds(i * B, B), 0:1]
        o_ref[...] = jnp.where(iota_r == iota_c + 1, esel, out)


@functools.partial(jax.jit, static_argnames=("interpret",))
def kernel(input_D, interpret=False):
    D = input_D[0]
    n = D.shape[0]
    g = n // B

    s, brow, bcol, ddr, ddc, dm1c, dm1r, e = pl.pallas_call(
        _cumsum_kernel,
        grid=(g, g),
        in_specs=[pl.BlockSpec((B, B), lambda i, j: (i, j))],
        out_specs=[
            pl.BlockSpec((B, B), lambda i, j: (i, j)),
            pl.BlockSpec((8 * (g + 1), n), lambda i, j: (0, 0)),
            pl.BlockSpec((n, NLANE), lambda i, j: (0, 0)),
            pl.BlockSpec((1, n), lambda i, j: (0, 0)),
            pl.BlockSpec((n, 1), lambda i, j: (0, 0)),
            pl.BlockSpec((n, 1), lambda i, j: (0, 0)),
            pl.BlockSpec((1, n), lambda i, j: (0, 0)),
            pl.BlockSpec((n, 1), lambda i, j: (0, 0)),
        ],
        out_shape=[
            jax.ShapeDtypeStruct((n, n), jnp.float32),
            jax.ShapeDtypeStruct((8 * (g + 1), n), jnp.float32),
            jax.ShapeDtypeStruct((n, NLANE), jnp.float32),
            jax.ShapeDtypeStruct((1, n), jnp.float32),
            jax.ShapeDtypeStruct((n, 1), jnp.float32),
            jax.ShapeDtypeStruct((n, 1), jnp.float32),
            jax.ShapeDtypeStruct((1, n), jnp.float32),
            jax.ShapeDtypeStruct((n, 1), jnp.float32),
        ],
        scratch_shapes=[
            pltpu.VMEM((B, 1), jnp.float32),
            pltpu.VMEM((1, n), jnp.float32),
            pltpu.VMEM((1, 1), jnp.float32),
            pltpu.VMEM((1, 1), jnp.float32),
        ],
        compiler_params=pltpu.CompilerParams(
            dimension_semantics=("arbitrary", "arbitrary")),
        interpret=interpret,
    )(D)

    out = pl.pallas_call(
        _out_kernel,
        grid=(g, g),
        in_specs=[
            pl.BlockSpec((B, B), lambda i, j: (jnp.minimum(i, j),
                                               jnp.maximum(i, j))),
            pl.BlockSpec((B, B), lambda i, j: (jnp.maximum(i, j),
                                               jnp.minimum(i, j))),
            pl.BlockSpec((8 * (g + 1), n), lambda i, j: (0, 0)),
            pl.BlockSpec((n, NLANE), lambda i, j: (0, 0)),
            pl.BlockSpec((1, n), lambda i, j: (0, 0)),
            pl.BlockSpec((n, 1), lambda i, j: (0, 0)),
            pl.BlockSpec((n, 1), lambda i, j: (0, 0)),
            pl.BlockSpec((1, n), lambda i, j: (0, 0)),
            pl.BlockSpec((n, 1), lambda i, j: (0, 0)),
        ],
        out_specs=pl.BlockSpec((B, B), lambda i, j: (i, j)),
        out_shape=jax.ShapeDtypeStruct((n, n), jnp.float32),
        compiler_params=pltpu.CompilerParams(
            dimension_semantics=("arbitrary", "arbitrary")),
        interpret=interpret,
    )(s, s, brow, bcol, ddr, ddc, dm1c, dm1r, e)

    return out[None, :, :]
